# Initial kernel scaffold; baseline (speedup 1.0000x reference)
import jax
import jax.numpy as jnp
from jax.experimental import pallas as pl


def kernel(x, edge_index, Wl1, bl1, Wr1, Wl2, bl2, Wr2, Wl3, bl3, Wr3):
    raise NotImplementedError("write your pallas kernel here")


# speedup vs baseline: 6.6618x; 6.6618x over previous
import functools

import jax
import jax.numpy as jnp
from jax import lax
from jax.experimental import pallas as pl
from jax.experimental.pallas import tpu as pltpu
from jax.experimental.pallas import tpu_sc as plsc

_N = 10000
_E = 320000
_D = 128
_NC = 2
_NS = 16
_NW = _NC * _NS
_EW = _E // _NW
_CHUNK = 80
_NCHUNK = _EW // _CHUNK
_CNTW = 128
_NP = 10240
_RPT = _NP // _NS


def _mesh():
    return plsc.VectorSubcoreMesh(
        core_axis_name="c", subcore_axis_name="s",
        num_cores=_NC, num_subcores=_NS)


def _worker_ids():
    c = lax.axis_index("c")
    s = lax.axis_index("s")
    return c, s, c * _NS + s


def _sc_agg_body(h_hbm, src_hbm, dst_hbm, z128_hbm, out_hbm,
                 src_v, dst_v, rows_v, acc_sh, sem):
    c, s, wid = _worker_ids()
    row0 = s * _RPT
    pltpu.sync_copy(z128_hbm, acc_sh.at[pl.ds(row0, _RPT)])
    pltpu.sync_copy(src_hbm.at[wid], src_v)
    pltpu.sync_copy(dst_hbm.at[wid], dst_v)
    plsc.subcore_barrier()

    def body(j, carry):
        pltpu.async_copy(h_hbm.at[src_v.at[j]], rows_v, sem).wait()
        pltpu.sync_copy(rows_v, acc_sh.at[dst_v.at[j]], add=True)
        return carry

    lax.fori_loop(0, _NCHUNK, body, 0)
    plsc.subcore_barrier()
    pltpu.sync_copy(acc_sh.at[pl.ds(row0, _RPT)],
                    out_hbm.at[c, pl.ds(row0, _RPT)])


def _sc_cnt_body(dst_hbm, z128_hbm, ones_hbm, cnt_out_hbm,
                 dst_v, ones_v, cnt_sh):
    c, s, wid = _worker_ids()
    row0 = s * _RPT
    pltpu.sync_copy(z128_hbm, cnt_sh.at[pl.ds(row0, _RPT)])
    pltpu.sync_copy(ones_hbm, ones_v)
    pltpu.sync_copy(dst_hbm.at[wid], dst_v)
    plsc.subcore_barrier()

    def body(j, carry):
        pltpu.sync_copy(ones_v, cnt_sh.at[dst_v.at[j]], add=True)
        return carry

    lax.fori_loop(0, _NCHUNK, body, 0)
    plsc.subcore_barrier()
    pltpu.sync_copy(cnt_sh.at[pl.ds(row0, _RPT)],
                    cnt_out_hbm.at[c, pl.ds(row0, _RPT)])


def _make_agg(interpret=False):
    return pl.kernel(
        _sc_agg_body,
        out_type=jax.ShapeDtypeStruct((_NC, _NP, _D), jnp.float32),
        mesh=_mesh(),
        scratch_types=[
            pltpu.VMEM((_NCHUNK, _CHUNK), jnp.int32),
            pltpu.VMEM((_NCHUNK, _CHUNK), jnp.int32),
            pltpu.VMEM((_CHUNK, _D), jnp.float32),
            pltpu.VMEM_SHARED((_NP, _D), jnp.float32),
            pltpu.SemaphoreType.DMA,
        ],
        interpret=interpret,
    )


def _make_cnt(interpret=False):
    return pl.kernel(
        _sc_cnt_body,
        out_type=jax.ShapeDtypeStruct((_NC, _NP, _CNTW), jnp.float32),
        mesh=_mesh(),
        scratch_types=[
            pltpu.VMEM((_NCHUNK, _CHUNK), jnp.int32),
            pltpu.VMEM((_CHUNK, _CNTW), jnp.float32),
            pltpu.VMEM_SHARED((_NP, _CNTW), jnp.float32),
        ],
        interpret=interpret,
    )


def _dense_body(relu, p_ref, c_ref, h_ref, wl_ref, wr_ref, bl_ref, o_ref):
    ssum = p_ref[0] + p_ref[1]
    cnt = c_ref[0, :, 0:1] + c_ref[1, :, 0:1]
    mean = ssum / jnp.maximum(cnt, 1.0)
    acc = lax.dot_general(mean, wl_ref[...], (((1,), (1,)), ((), ())),
                          preferred_element_type=jnp.float32)
    acc = acc + lax.dot_general(h_ref[...], wr_ref[...], (((1,), (1,)), ((), ())),
                                preferred_element_type=jnp.float32)
    acc = acc + bl_ref[...]
    o_ref[...] = jnp.maximum(acc, 0.0) if relu else acc


def _dense(part, cnt, h, Wl, bl, Wr, relu, interpret=False):
    bm = 512
    return pl.pallas_call(
        functools.partial(_dense_body, relu),
        grid=(_NP // bm,),
        in_specs=[
            pl.BlockSpec((_NC, bm, _D), lambda i: (0, i, 0)),
            pl.BlockSpec((_NC, bm, _CNTW), lambda i: (0, i, 0)),
            pl.BlockSpec((bm, _D), lambda i: (i, 0)),
            pl.BlockSpec((_D, _D), lambda i: (0, 0)),
            pl.BlockSpec((_D, _D), lambda i: (0, 0)),
            pl.BlockSpec((1, _D), lambda i: (0, 0)),
        ],
        out_specs=pl.BlockSpec((bm, _D), lambda i: (i, 0)),
        out_shape=jax.ShapeDtypeStruct((_NP, _D), jnp.float32),
        interpret=interpret,
    )(part, cnt, h, Wl, Wr, bl.reshape(1, _D))


def kernel(x, edge_index, Wl1, bl1, Wr1, Wl2, bl2, Wr2, Wl3, bl3, Wr3):
    xp = jnp.pad(x, ((0, _NP - _N), (0, 0)))
    src = edge_index[0].astype(jnp.int32).reshape(_NW, _NCHUNK, _CHUNK)
    dst = edge_index[1].astype(jnp.int32).reshape(_NW, _NCHUNK, _CHUNK)
    z128 = jnp.zeros((_RPT, _D), jnp.float32)
    ones = jnp.ones((_CHUNK, _CNTW), jnp.float32)

    agg = _make_agg()
    cnt = _make_cnt()(dst, z128, ones)

    part1 = agg(xp, src, dst, z128)
    h1 = _dense(part1, cnt, xp, Wl1, bl1, Wr1, True)
    part2 = agg(h1, src, dst, z128)
    h2 = _dense(part2, cnt, h1, Wl2, bl2, Wr2, True)
    part3 = agg(h2, src, dst, z128)
    out = _dense(part3, cnt, h2, Wl3, bl3, Wr3, False)
    return out[:_N]
